# baseline (speedup 1.0000x reference)
import functools

import jax
import jax.numpy as jnp
from jax import lax
from jax.experimental import pallas as pl
from jax.experimental.pallas import tpu as pltpu
from jax.experimental.pallas import tpu_sc as plsc

N = 10000
NP = 10240
E = 320000
EP = 327680
H = 128

NC = 2
NS = 16
NW = NC * NS
PER_W = EP // NW
CHUNK = 128
NCH = PER_W // CHUNK
RSUB = NP // NS

K = 2
EPK = EP // K
PER_WK = EPK // NW
NCHK = PER_WK // CHUNK

BN = 1024
BE = 2048


def _dot(a, w):
    return jnp.dot(a, w, preferred_element_type=jnp.float32)


def _silu(v):
    return v * jax.nn.sigmoid(v)


def _ln(v, g, b):
    mu = jnp.mean(v, axis=-1, keepdims=True)
    d = v - mu
    var = jnp.mean(d * d, axis=-1, keepdims=True)
    return d * lax.rsqrt(var + 1e-5) * g + b


def _full(shape):
    return pl.BlockSpec(shape, lambda i: tuple(0 for _ in shape))


def _rows(nrows, width):
    return pl.BlockSpec((nrows, width), lambda i: (i, 0))


def _mlp_args(p, with_ln):
    args = [p["l0"]["W"], p["l0"]["b"].reshape(1, -1),
            p["l1"]["W"], p["l1"]["b"].reshape(1, -1),
            p["l2"]["W"], p["l2"]["b"].reshape(1, -1)]
    if with_ln:
        args += [p["ln"]["g"].reshape(1, -1), p["ln"]["b"].reshape(1, -1)]
    return args


def _mlp_specs(fin, with_ln):
    specs = [_full((fin, H)), _full((1, H)), _full((H, H)), _full((1, H)),
             _full((H, H)), _full((1, H))]
    if with_ln:
        specs += [_full((1, H)), _full((1, H))]
    return specs


def _node_enc_call(xp, ne, w0i, w0j, b0e):
    def body(x_ref, w0, b0, w1, b1, w2, b2, g, bl, wi, wj, be0,
             x1_ref, a_ref, b_ref):
        h = _silu(_dot(x_ref[...], w0[...]) + b0[...])
        h = _silu(_dot(h, w1[...]) + b1[...])
        h = _dot(h, w2[...]) + b2[...]
        x1 = _ln(h, g[...], bl[...])
        x1_ref[...] = x1
        a_ref[...] = _dot(x1, wi[...]) + be0[...]
        b_ref[...] = _dot(x1, wj[...])

    return pl.pallas_call(
        body,
        grid=(NP // BN,),
        in_specs=[_rows(BN, H)] + _mlp_specs(H, True)
        + [_full((H, H)), _full((H, H)), _full((1, H))],
        out_specs=[_rows(BN, H)] * 3,
        out_shape=[jax.ShapeDtypeStruct((NP, H), jnp.float32)] * 3,
    )(xp, *_mlp_args(ne, True), w0i, w0j, b0e.reshape(1, H))


def _edge_enc_call(eap, ee):
    def body(e_ref, w0, b0, w1, b1, w2, b2, g, bl, out_ref):
        h = _silu(_dot(e_ref[...], w0[...]) + b0[...])
        h = _silu(_dot(h, w1[...]) + b1[...])
        h = _dot(h, w2[...]) + b2[...]
        out_ref[...] = _ln(h, g[...], bl[...])

    return pl.pallas_call(
        body,
        grid=(EPK // BE,),
        in_specs=[_rows(BE, 16)] + _mlp_specs(16, True),
        out_specs=_rows(BE, H),
        out_shape=jax.ShapeDtypeStruct((EPK, H), jnp.float32),
    )(eap, *_mlp_args(ee, True))


def _edge_layer_call(gab, gbb, e, pe):
    def body(ga, gb, e_ref, w0e, w1, b1, w2, b2, g, bl, out_ref):
        ev = e_ref[...]
        h = ga[...] + gb[...] + _dot(ev, w0e[...])
        h = _silu(h)
        h = _silu(_dot(h, w1[...]) + b1[...])
        h = _dot(h, w2[...]) + b2[...]
        out_ref[...] = _ln(h, g[...], bl[...]) + ev

    w0e = pe["l0"]["W"][2 * H:]
    return pl.pallas_call(
        body,
        grid=(EPK // BE,),
        in_specs=[_rows(BE, H), _rows(BE, H), _rows(BE, H), _full((H, H)),
                  _full((H, H)), _full((1, H)), _full((H, H)), _full((1, H)),
                  _full((1, H)), _full((1, H))],
        out_specs=_rows(BE, H),
        out_shape=jax.ShapeDtypeStruct((EPK, H), jnp.float32),
    )(gab, gbb, e, w0e,
      pe["l1"]["W"], pe["l1"]["b"].reshape(1, H),
      pe["l2"]["W"], pe["l2"]["b"].reshape(1, H),
      pe["ln"]["g"].reshape(1, H), pe["ln"]["b"].reshape(1, H))


def _node_layer_call(x1, s2, pn, w0i, w0j, b0e):
    def body(x_ref, s0, s1, w0x, w0a, b0, w1, b1, w2, b2, g, bl,
             wi, wj, be0, x_out, a_out, b_out):
        x = x_ref[...]
        agg = s0[...] + s1[...]
        h = _silu(_dot(x, w0x[...]) + _dot(agg, w0a[...]) + b0[...])
        h = _silu(_dot(h, w1[...]) + b1[...])
        h = _dot(h, w2[...]) + b2[...]
        xn = x + _ln(h, g[...], bl[...])
        x_out[...] = xn
        a_out[...] = _dot(xn, wi[...]) + be0[...]
        b_out[...] = _dot(xn, wj[...])

    w0 = pn["l0"]["W"]
    nb = NP // BN
    return pl.pallas_call(
        body,
        grid=(nb,),
        in_specs=[_rows(BN, H),
                  pl.BlockSpec((BN, H), lambda i: (i, 0)),
                  pl.BlockSpec((BN, H), lambda i: (i + nb, 0)),
                  _full((H, H)), _full((H, H)), _full((1, H)),
                  _full((H, H)), _full((1, H)), _full((H, H)), _full((1, H)),
                  _full((1, H)), _full((1, H)),
                  _full((H, H)), _full((H, H)), _full((1, H))],
        out_specs=[_rows(BN, H)] * 3,
        out_shape=[jax.ShapeDtypeStruct((NP, H), jnp.float32)] * 3,
    )(x1, s2, s2, w0[:H], w0[H:],
      pn["l0"]["b"].reshape(1, H),
      pn["l1"]["W"], pn["l1"]["b"].reshape(1, H),
      pn["l2"]["W"], pn["l2"]["b"].reshape(1, H),
      pn["ln"]["g"].reshape(1, H), pn["ln"]["b"].reshape(1, H),
      w0i, w0j, b0e.reshape(1, H))


def _final_call(x1, s2, pn, dec):
    def body(x_ref, s0, s1, w0x, w0a, b0, w1, b1, w2, b2, g, bl,
             d0, db0, d1, db1, d2, db2, out_ref):
        x = x_ref[...]
        agg = s0[...] + s1[...]
        h = _silu(_dot(x, w0x[...]) + _dot(agg, w0a[...]) + b0[...])
        h = _silu(_dot(h, w1[...]) + b1[...])
        h = _dot(h, w2[...]) + b2[...]
        xn = x + _ln(h, g[...], bl[...])
        h = _silu(_dot(xn, d0[...]) + db0[...])
        h = _silu(_dot(h, d1[...]) + db1[...])
        out_ref[...] = _dot(h, d2[...]) + db2[...]

    w0 = pn["l0"]["W"]
    nb = NP // BN
    return pl.pallas_call(
        body,
        grid=(nb,),
        in_specs=[_rows(BN, H),
                  pl.BlockSpec((BN, H), lambda i: (i, 0)),
                  pl.BlockSpec((BN, H), lambda i: (i + nb, 0)),
                  _full((H, H)), _full((H, H)), _full((1, H)),
                  _full((H, H)), _full((1, H)), _full((H, H)), _full((1, H)),
                  _full((1, H)), _full((1, H))] + _mlp_specs(H, False),
        out_specs=_rows(BN, H),
        out_shape=jax.ShapeDtypeStruct((NP, H), jnp.float32),
    )(x1, s2, s2, w0[:H], w0[H:],
      pn["l0"]["b"].reshape(1, H),
      pn["l1"]["W"], pn["l1"]["b"].reshape(1, H),
      pn["l2"]["W"], pn["l2"]["b"].reshape(1, H),
      pn["ln"]["g"].reshape(1, H), pn["ln"]["b"].reshape(1, H),
      *_mlp_args(dec, False))


def _sc_gather(ai, bi, dstc, srcc):
    mesh = plsc.VectorSubcoreMesh(core_axis_name="c", subcore_axis_name="s")

    @functools.partial(
        pl.kernel,
        out_type=(jax.ShapeDtypeStruct((EPK, H), jnp.float32),) * 2,
        mesh=mesh,
        scratch_types=[
            pltpu.VMEM((NCHK, CHUNK), jnp.int32),
            pltpu.VMEM((NCHK, CHUNK), jnp.int32),
            pltpu.VMEM((2, CHUNK, H), jnp.float32),
            pltpu.VMEM((2, CHUNK, H), jnp.float32),
            pltpu.SemaphoreType.DMA,
            pltpu.SemaphoreType.DMA,
            pltpu.SemaphoreType.DMA,
            pltpu.SemaphoreType.DMA,
        ],
    )
    def k(a_hbm, b_hbm, d_hbm, s_hbm, ga_hbm, gb_hbm,
          d_v, s_v, bufa, bufb, semg, semg2, semwa, semwb):
        wid = lax.axis_index("s") * NC + lax.axis_index("c")
        pltpu.sync_copy(d_hbm.at[wid], d_v)
        pltpu.sync_copy(s_hbm.at[wid], s_v)
        base = wid * PER_WK

        def row(j):
            return ga_hbm.at[pl.ds(base + j * CHUNK, CHUNK)]

        def rowb(j):
            return gb_hbm.at[pl.ds(base + j * CHUNK, CHUNK)]

        @pl.loop(0, NCHK, step=2)
        def _(j0):
            for t in range(2):
                j = j0 + t

                @pl.when(j0 >= 2)
                def _():
                    pltpu.make_async_copy(bufa.at[t], row(j - 2),
                                          semwa).wait()
                    pltpu.make_async_copy(bufb.at[t], rowb(j - 2),
                                          semwb).wait()

                ca = pltpu.async_copy(a_hbm.at[d_v.at[j]], bufa.at[t], semg)
                cb = pltpu.async_copy(b_hbm.at[s_v.at[j]], bufb.at[t], semg2)
                ca.wait()
                pltpu.async_copy(bufa.at[t], row(j), semwa)
                cb.wait()
                pltpu.async_copy(bufb.at[t], rowb(j), semwb)

        for t in range(2):
            j = NCHK - 2 + t
            pltpu.make_async_copy(bufa.at[t], row(j), semwa).wait()
            pltpu.make_async_copy(bufb.at[t], rowb(j), semwb).wait()

    return k(ai, bi, dstc, srcc)


def _sc_scatter(e_list, srcc_list, zsub):
    mesh = plsc.VectorSubcoreMesh(core_axis_name="c", subcore_axis_name="s")

    @functools.partial(
        pl.kernel,
        out_type=jax.ShapeDtypeStruct((2 * NP, H), jnp.float32),
        mesh=mesh,
        scratch_types=[
            pltpu.VMEM_SHARED((NP, H), jnp.float32),
            pltpu.VMEM((NCHK, CHUNK), jnp.int32),
            pltpu.VMEM((NCHK, CHUNK), jnp.int32),
            pltpu.VMEM((2, CHUNK, H), jnp.float32),
            pltpu.SemaphoreType.DMA,
            pltpu.SemaphoreType.DMA,
        ],
    )
    def k(e0_hbm, e1_hbm, s0_hbm, s1_hbm, z_hbm, out_hbm,
          shared, s_v0, s_v1, buf, semr, sema):
        c = lax.axis_index("c")
        s = lax.axis_index("s")
        wid = s * NC + c
        pltpu.sync_copy(z_hbm, shared.at[pl.ds(s * RSUB, RSUB)])
        pltpu.sync_copy(s0_hbm.at[wid], s_v0)
        pltpu.sync_copy(s1_hbm.at[wid], s_v1)
        plsc.subcore_barrier()
        base = wid * PER_WK

        for e_hbm, s_v in ((e0_hbm, s_v0), (e1_hbm, s_v1)):

            @pl.loop(0, NCHK, step=2)
            def _(j0):
                for t in range(2):
                    j = j0 + t

                    @pl.when(j0 >= 2)
                    def _():
                        pltpu.make_async_copy(
                            buf.at[t], shared.at[s_v.at[j - 2]], sema).wait()

                    cr = pltpu.async_copy(
                        e_hbm.at[pl.ds(base + j * CHUNK, CHUNK)], buf.at[t],
                        semr)
                    cr.wait()
                    pltpu.async_copy(buf.at[t], shared.at[s_v.at[j]], sema,
                                     add=True)

            for t in range(2):
                j = NCHK - 2 + t
                pltpu.make_async_copy(buf.at[t], shared.at[s_v.at[j]],
                                      sema).wait()

        plsc.subcore_barrier()
        pltpu.sync_copy(shared.at[pl.ds(s * RSUB, RSUB)],
                        out_hbm.at[pl.ds(c * NP + s * RSUB, RSUB)])

    return k(e_list[0], e_list[1], srcc_list[0], srcc_list[1], zsub)


def kernel(x, edge_index, edge_attr, params):
    src = edge_index[0].astype(jnp.int32)
    dst = edge_index[1].astype(jnp.int32)
    pad_e = EP - E
    pad_idx = N + jnp.arange(pad_e, dtype=jnp.int32) % (NP - N)
    srcp = jnp.concatenate([src, pad_idx]).reshape(K, NW, NCHK, CHUNK)
    dstp = jnp.concatenate([dst, pad_idx]).reshape(K, NW, NCHK, CHUNK)
    srcc = [srcp[k] for k in range(K)]
    dstc = [dstp[k] for k in range(K)]
    xp = jnp.pad(x, ((0, NP - N), (0, 0)))
    eap = jnp.pad(edge_attr, ((0, pad_e), (0, 0))).reshape(K, EPK, 16)
    zsub = jnp.zeros((RSUB, H), jnp.float32)

    pr = params["proc"]
    w00 = pr[0]["edge_mlp"]["l0"]["W"]
    x1, at, bt = _node_enc_call(xp, params["node_enc"], w00[:H], w00[H:2 * H],
                                pr[0]["edge_mlp"]["l0"]["b"])
    e = [_edge_enc_call(eap[k], params["edge_enc"]) for k in range(K)]

    out = None
    for i in range(3):
        pe = pr[i]["edge_mlp"]
        for k in range(K):
            ga, gb = _sc_gather(at, bt, dstc[k], srcc[k])
            e[k] = _edge_layer_call(ga, gb, e[k], pe)
        s2 = _sc_scatter(e, srcc, zsub)
        if i < 2:
            w0n = pr[i + 1]["edge_mlp"]["l0"]["W"]
            x1, at, bt = _node_layer_call(
                x1, s2, pr[i]["node_mlp"], w0n[:H], w0n[H:2 * H],
                pr[i + 1]["edge_mlp"]["l0"]["b"])
        else:
            out = _final_call(x1, s2, pr[i]["node_mlp"], params["decoder"])
    return out[:N]


# speedup vs baseline: 1.1137x; 1.1137x over previous
import functools

import jax
import jax.numpy as jnp
from jax import lax
from jax.experimental import pallas as pl
from jax.experimental.pallas import tpu as pltpu
from jax.experimental.pallas import tpu_sc as plsc

N = 10000
NP = 10240
E = 320000
EP = 327680
H = 128

NC = 2
NS = 16
NW = NC * NS
PER_W = EP // NW
CHUNK = 128
NCH = PER_W // CHUNK
RSUB = NP // NS

K = 2
EPK = EP // K
PER_WK = EPK // NW
NCHK = PER_WK // CHUNK

BN = 1024
BE = 2048


def _dot(a, w):
    return jnp.dot(a, w, preferred_element_type=jnp.float32)


def _silu(v):
    return v * jax.nn.sigmoid(v)


def _ln(v, g, b):
    mu = jnp.mean(v, axis=-1, keepdims=True)
    d = v - mu
    var = jnp.mean(d * d, axis=-1, keepdims=True)
    return d * lax.rsqrt(var + 1e-5) * g + b


def _full(shape):
    return pl.BlockSpec(shape, lambda i: tuple(0 for _ in shape))


def _rows(nrows, width):
    return pl.BlockSpec((nrows, width), lambda i: (i, 0))


def _mlp_args(p, with_ln):
    args = [p["l0"]["W"], p["l0"]["b"].reshape(1, -1),
            p["l1"]["W"], p["l1"]["b"].reshape(1, -1),
            p["l2"]["W"], p["l2"]["b"].reshape(1, -1)]
    if with_ln:
        args += [p["ln"]["g"].reshape(1, -1), p["ln"]["b"].reshape(1, -1)]
    return args


def _mlp_specs(fin, with_ln):
    specs = [_full((fin, H)), _full((1, H)), _full((H, H)), _full((1, H)),
             _full((H, H)), _full((1, H))]
    if with_ln:
        specs += [_full((1, H)), _full((1, H))]
    return specs


def _node_enc_call(xp, ne, w0i, w0j, b0e):
    def body(x_ref, w0, b0, w1, b1, w2, b2, g, bl, wi, wj, be0,
             x1_ref, a_ref, b_ref):
        h = _silu(_dot(x_ref[...], w0[...]) + b0[...])
        h = _silu(_dot(h, w1[...]) + b1[...])
        h = _dot(h, w2[...]) + b2[...]
        x1 = _ln(h, g[...], bl[...])
        x1_ref[...] = x1
        a_ref[...] = _dot(x1, wi[...]) + be0[...]
        b_ref[...] = _dot(x1, wj[...])

    return pl.pallas_call(
        body,
        grid=(NP // BN,),
        in_specs=[_rows(BN, H)] + _mlp_specs(H, True)
        + [_full((H, H)), _full((H, H)), _full((1, H))],
        out_specs=[_rows(BN, H)] * 3,
        out_shape=[jax.ShapeDtypeStruct((NP, H), jnp.float32)] * 3,
    )(xp, *_mlp_args(ne, True), w0i, w0j, b0e.reshape(1, H))


def _edge_enc_call(eap, ee):
    def body(e_ref, w0, b0, w1, b1, w2, b2, g, bl, out_ref):
        h = _silu(_dot(e_ref[...], w0[...]) + b0[...])
        h = _silu(_dot(h, w1[...]) + b1[...])
        h = _dot(h, w2[...]) + b2[...]
        out_ref[...] = _ln(h, g[...], bl[...])

    return pl.pallas_call(
        body,
        grid=(EPK // BE,),
        in_specs=[_rows(BE, 16)] + _mlp_specs(16, True),
        out_specs=_rows(BE, H),
        out_shape=jax.ShapeDtypeStruct((EPK, H), jnp.float32),
    )(eap, *_mlp_args(ee, True))


def _edge_layer_call(gab, gbb, e, pe):
    def body(ga, gb, e_ref, w0e, w1, b1, w2, b2, g, bl, out_ref):
        ev = e_ref[...]
        h = ga[...] + gb[...] + _dot(ev, w0e[...])
        h = _silu(h)
        h = _silu(_dot(h, w1[...]) + b1[...])
        h = _dot(h, w2[...]) + b2[...]
        out_ref[...] = _ln(h, g[...], bl[...]) + ev

    w0e = pe["l0"]["W"][2 * H:]
    return pl.pallas_call(
        body,
        grid=(EPK // BE,),
        in_specs=[_rows(BE, H), _rows(BE, H), _rows(BE, H), _full((H, H)),
                  _full((H, H)), _full((1, H)), _full((H, H)), _full((1, H)),
                  _full((1, H)), _full((1, H))],
        out_specs=_rows(BE, H),
        out_shape=jax.ShapeDtypeStruct((EPK, H), jnp.float32),
    )(gab, gbb, e, w0e,
      pe["l1"]["W"], pe["l1"]["b"].reshape(1, H),
      pe["l2"]["W"], pe["l2"]["b"].reshape(1, H),
      pe["ln"]["g"].reshape(1, H), pe["ln"]["b"].reshape(1, H))


def _edge_layer0_call(gab, gbb, eap, ee, pe):
    def body(ga, gb, ea_ref, w0, b0, w1, b1, w2, b2, g, bl,
             pw0e, pw1, pb1, pw2, pb2, pg, pbl, out_ref):
        he = _silu(_dot(ea_ref[...], w0[...]) + b0[...])
        he = _silu(_dot(he, w1[...]) + b1[...])
        he = _dot(he, w2[...]) + b2[...]
        ev = _ln(he, g[...], bl[...])
        h = ga[...] + gb[...] + _dot(ev, pw0e[...])
        h = _silu(h)
        h = _silu(_dot(h, pw1[...]) + pb1[...])
        h = _dot(h, pw2[...]) + pb2[...]
        out_ref[...] = _ln(h, pg[...], pbl[...]) + ev

    w0e = pe["l0"]["W"][2 * H:]
    return pl.pallas_call(
        body,
        grid=(EPK // BE,),
        in_specs=[_rows(BE, H), _rows(BE, H), _rows(BE, 16)]
        + _mlp_specs(16, True)
        + [_full((H, H)), _full((H, H)), _full((1, H)), _full((H, H)),
           _full((1, H)), _full((1, H)), _full((1, H))],
        out_specs=_rows(BE, H),
        out_shape=jax.ShapeDtypeStruct((EPK, H), jnp.float32),
    )(gab, gbb, eap, *_mlp_args(ee, True), w0e,
      pe["l1"]["W"], pe["l1"]["b"].reshape(1, H),
      pe["l2"]["W"], pe["l2"]["b"].reshape(1, H),
      pe["ln"]["g"].reshape(1, H), pe["ln"]["b"].reshape(1, H))


def _node_layer_call(x1, s2, pn, w0i, w0j, b0e):
    def body(x_ref, s0, s1, s2, s3, w0x, w0a, b0, w1, b1, w2, b2, g, bl,
             wi, wj, be0, x_out, a_out, b_out):
        x = x_ref[...]
        agg = (s0[...] + s1[...]) + (s2[...] + s3[...])
        h = _silu(_dot(x, w0x[...]) + _dot(agg, w0a[...]) + b0[...])
        h = _silu(_dot(h, w1[...]) + b1[...])
        h = _dot(h, w2[...]) + b2[...]
        xn = x + _ln(h, g[...], bl[...])
        x_out[...] = xn
        a_out[...] = _dot(xn, wi[...]) + be0[...]
        b_out[...] = _dot(xn, wj[...])

    w0 = pn["l0"]["W"]
    nb = NP // BN
    return pl.pallas_call(
        body,
        grid=(nb,),
        in_specs=[_rows(BN, H),
                  pl.BlockSpec((BN, H), lambda i: (i, 0)),
                  pl.BlockSpec((BN, H), lambda i: (i + nb, 0)),
                  pl.BlockSpec((BN, H), lambda i: (i, 0)),
                  pl.BlockSpec((BN, H), lambda i: (i + nb, 0)),
                  _full((H, H)), _full((H, H)), _full((1, H)),
                  _full((H, H)), _full((1, H)), _full((H, H)), _full((1, H)),
                  _full((1, H)), _full((1, H)),
                  _full((H, H)), _full((H, H)), _full((1, H))],
        out_specs=[_rows(BN, H)] * 3,
        out_shape=[jax.ShapeDtypeStruct((NP, H), jnp.float32)] * 3,
    )(x1, s2[0], s2[0], s2[1], s2[1], w0[:H], w0[H:],
      pn["l0"]["b"].reshape(1, H),
      pn["l1"]["W"], pn["l1"]["b"].reshape(1, H),
      pn["l2"]["W"], pn["l2"]["b"].reshape(1, H),
      pn["ln"]["g"].reshape(1, H), pn["ln"]["b"].reshape(1, H),
      w0i, w0j, b0e.reshape(1, H))


def _final_call(x1, s2, pn, dec):
    def body(x_ref, s0, s1, s2, s3, w0x, w0a, b0, w1, b1, w2, b2, g, bl,
             d0, db0, d1, db1, d2, db2, out_ref):
        x = x_ref[...]
        agg = (s0[...] + s1[...]) + (s2[...] + s3[...])
        h = _silu(_dot(x, w0x[...]) + _dot(agg, w0a[...]) + b0[...])
        h = _silu(_dot(h, w1[...]) + b1[...])
        h = _dot(h, w2[...]) + b2[...]
        xn = x + _ln(h, g[...], bl[...])
        h = _silu(_dot(xn, d0[...]) + db0[...])
        h = _silu(_dot(h, d1[...]) + db1[...])
        out_ref[...] = _dot(h, d2[...]) + db2[...]

    w0 = pn["l0"]["W"]
    nb = NP // BN
    return pl.pallas_call(
        body,
        grid=(nb,),
        in_specs=[_rows(BN, H),
                  pl.BlockSpec((BN, H), lambda i: (i, 0)),
                  pl.BlockSpec((BN, H), lambda i: (i + nb, 0)),
                  pl.BlockSpec((BN, H), lambda i: (i, 0)),
                  pl.BlockSpec((BN, H), lambda i: (i + nb, 0)),
                  _full((H, H)), _full((H, H)), _full((1, H)),
                  _full((H, H)), _full((1, H)), _full((H, H)), _full((1, H)),
                  _full((1, H)), _full((1, H))] + _mlp_specs(H, False),
        out_specs=_rows(BN, H),
        out_shape=jax.ShapeDtypeStruct((NP, H), jnp.float32),
    )(x1, s2[0], s2[0], s2[1], s2[1], w0[:H], w0[H:],
      pn["l0"]["b"].reshape(1, H),
      pn["l1"]["W"], pn["l1"]["b"].reshape(1, H),
      pn["l2"]["W"], pn["l2"]["b"].reshape(1, H),
      pn["ln"]["g"].reshape(1, H), pn["ln"]["b"].reshape(1, H),
      *_mlp_args(dec, False))


def _sc_gather(ai, bi, dstc, srcc):
    mesh = plsc.VectorSubcoreMesh(core_axis_name="c", subcore_axis_name="s")

    @functools.partial(
        pl.kernel,
        out_type=(jax.ShapeDtypeStruct((EPK, H), jnp.float32),) * 2,
        mesh=mesh,
        scratch_types=[
            pltpu.VMEM((NCHK, CHUNK), jnp.int32),
            pltpu.VMEM((NCHK, CHUNK), jnp.int32),
            pltpu.VMEM((2, CHUNK, H), jnp.float32),
            pltpu.VMEM((2, CHUNK, H), jnp.float32),
            pltpu.SemaphoreType.DMA,
            pltpu.SemaphoreType.DMA,
            pltpu.SemaphoreType.DMA,
            pltpu.SemaphoreType.DMA,
        ],
    )
    def k(a_hbm, b_hbm, d_hbm, s_hbm, ga_hbm, gb_hbm,
          d_v, s_v, bufa, bufb, semg, semg2, semwa, semwb):
        wid = lax.axis_index("s") * NC + lax.axis_index("c")
        pltpu.sync_copy(d_hbm.at[wid], d_v)
        pltpu.sync_copy(s_hbm.at[wid], s_v)
        base = wid * PER_WK

        def row(j):
            return ga_hbm.at[pl.ds(base + j * CHUNK, CHUNK)]

        def rowb(j):
            return gb_hbm.at[pl.ds(base + j * CHUNK, CHUNK)]

        @pl.loop(0, NCHK, step=2)
        def _(j0):
            for t in range(2):
                j = j0 + t

                @pl.when(j0 >= 2)
                def _():
                    pltpu.make_async_copy(bufa.at[t], row(j - 2),
                                          semwa).wait()
                    pltpu.make_async_copy(bufb.at[t], rowb(j - 2),
                                          semwb).wait()

                ca = pltpu.async_copy(a_hbm.at[d_v.at[j]], bufa.at[t], semg)
                cb = pltpu.async_copy(b_hbm.at[s_v.at[j]], bufb.at[t], semg2)
                ca.wait()
                pltpu.async_copy(bufa.at[t], row(j), semwa)
                cb.wait()
                pltpu.async_copy(bufb.at[t], rowb(j), semwb)

        for t in range(2):
            j = NCHK - 2 + t
            pltpu.make_async_copy(bufa.at[t], row(j), semwa).wait()
            pltpu.make_async_copy(bufb.at[t], rowb(j), semwb).wait()

    return k(ai, bi, dstc, srcc)


def _sc_scatter(e_new, srcc, zsub):
    mesh = plsc.VectorSubcoreMesh(core_axis_name="c", subcore_axis_name="s")

    @functools.partial(
        pl.kernel,
        out_type=jax.ShapeDtypeStruct((2 * NP, H), jnp.float32),
        mesh=mesh,
        scratch_types=[
            pltpu.VMEM_SHARED((NP, H), jnp.float32),
            pltpu.VMEM((NCHK, CHUNK), jnp.int32),
            pltpu.VMEM((2, CHUNK, H), jnp.float32),
            pltpu.SemaphoreType.DMA,
            pltpu.SemaphoreType.DMA,
        ],
    )
    def k(e_hbm, s_hbm, z_hbm, out_hbm, shared, s_v, buf, semr, sema):
        c = lax.axis_index("c")
        s = lax.axis_index("s")
        wid = s * NC + c
        pltpu.sync_copy(z_hbm, shared.at[pl.ds(s * RSUB, RSUB)])
        pltpu.sync_copy(s_hbm.at[wid], s_v)
        plsc.subcore_barrier()
        base = wid * PER_WK

        @pl.loop(0, NCHK, step=2)
        def _(j0):
            for t in range(2):
                j = j0 + t

                @pl.when(j0 >= 2)
                def _():
                    pltpu.make_async_copy(buf.at[t], shared.at[s_v.at[j - 2]],
                                          sema).wait()

                cr = pltpu.async_copy(
                    e_hbm.at[pl.ds(base + j * CHUNK, CHUNK)], buf.at[t], semr)
                cr.wait()
                pltpu.async_copy(buf.at[t], shared.at[s_v.at[j]], sema,
                                 add=True)

        for t in range(2):
            j = NCHK - 2 + t
            pltpu.make_async_copy(buf.at[t], shared.at[s_v.at[j]], sema).wait()

        plsc.subcore_barrier()
        pltpu.sync_copy(shared.at[pl.ds(s * RSUB, RSUB)],
                        out_hbm.at[pl.ds(c * NP + s * RSUB, RSUB)])

    return k(e_new, srcc, zsub)


def kernel(x, edge_index, edge_attr, params):
    src = edge_index[0].astype(jnp.int32)
    dst = edge_index[1].astype(jnp.int32)
    pad_e = EP - E
    pad_idx = N + jnp.arange(pad_e, dtype=jnp.int32) % (NP - N)
    srcp = jnp.concatenate([src, pad_idx]).reshape(K, NW, NCHK, CHUNK)
    dstp = jnp.concatenate([dst, pad_idx]).reshape(K, NW, NCHK, CHUNK)
    srcc = [srcp[k] for k in range(K)]
    dstc = [dstp[k] for k in range(K)]
    xp = jnp.pad(x, ((0, NP - N), (0, 0)))
    eap = jnp.pad(edge_attr, ((0, pad_e), (0, 0))).reshape(K, EPK, 16)
    zsub = jnp.zeros((RSUB, H), jnp.float32)

    pr = params["proc"]
    w00 = pr[0]["edge_mlp"]["l0"]["W"]
    x1, at, bt = _node_enc_call(xp, params["node_enc"], w00[:H], w00[H:2 * H],
                                pr[0]["edge_mlp"]["l0"]["b"])
    e = [None] * K

    out = None
    for i in range(3):
        pe = pr[i]["edge_mlp"]
        s2 = []
        for k in range(K):
            ga, gb = _sc_gather(at, bt, dstc[k], srcc[k])
            if i == 0:
                e[k] = _edge_layer0_call(ga, gb, eap[k], params["edge_enc"],
                                         pe)
            else:
                e[k] = _edge_layer_call(ga, gb, e[k], pe)
            s2.append(_sc_scatter(e[k], srcc[k], zsub))
        if i < 2:
            w0n = pr[i + 1]["edge_mlp"]["l0"]["W"]
            x1, at, bt = _node_layer_call(
                x1, s2, pr[i]["node_mlp"], w0n[:H], w0n[H:2 * H],
                pr[i + 1]["edge_mlp"]["l0"]["b"])
        else:
            out = _final_call(x1, s2, pr[i]["node_mlp"], params["decoder"])
    return out[:N]


# speedup vs baseline: 1.2909x; 1.1591x over previous
import functools

import jax
import jax.numpy as jnp
from jax import lax
from jax.experimental import pallas as pl
from jax.experimental.pallas import tpu as pltpu
from jax.experimental.pallas import tpu_sc as plsc

N = 10000
NP = 10240
E = 320000
EP = 327680
H = 128

NC = 2
NS = 16
NW = NC * NS
PER_W = EP // NW
CHUNK = 128
NCH = PER_W // CHUNK
RSUB = NP // NS

K = 2
EPK = EP // K
PER_WK = EPK // NW
NCHK = PER_WK // CHUNK

BN = 1024
BE = 2048


def _dot(a, w):
    return jnp.dot(a, w, preferred_element_type=jnp.float32)


def _silu(v):
    return v * jax.nn.sigmoid(v)


def _ln(v, g, b):
    mu = jnp.mean(v, axis=-1, keepdims=True)
    d = v - mu
    var = jnp.mean(d * d, axis=-1, keepdims=True)
    return d * lax.rsqrt(var + 1e-5) * g + b


def _full(shape):
    return pl.BlockSpec(shape, lambda i: tuple(0 for _ in shape))


def _rows(nrows, width):
    return pl.BlockSpec((nrows, width), lambda i: (i, 0))


def _mlp_args(p, with_ln):
    args = [p["l0"]["W"], p["l0"]["b"].reshape(1, -1),
            p["l1"]["W"], p["l1"]["b"].reshape(1, -1),
            p["l2"]["W"], p["l2"]["b"].reshape(1, -1)]
    if with_ln:
        args += [p["ln"]["g"].reshape(1, -1), p["ln"]["b"].reshape(1, -1)]
    return args


def _mlp_specs(fin, with_ln):
    specs = [_full((fin, H)), _full((1, H)), _full((H, H)), _full((1, H)),
             _full((H, H)), _full((1, H))]
    if with_ln:
        specs += [_full((1, H)), _full((1, H))]
    return specs


def _node_enc_call(xp, ne, w0i, w0j, b0e):
    def body(x_ref, w0, b0, w1, b1, w2, b2, g, bl, wi, wj, be0,
             x1_ref, a_ref, b_ref):
        h = _silu(_dot(x_ref[...], w0[...]) + b0[...])
        h = _silu(_dot(h, w1[...]) + b1[...])
        h = _dot(h, w2[...]) + b2[...]
        x1 = _ln(h, g[...], bl[...])
        x1_ref[...] = x1
        a_ref[...] = _dot(x1, wi[...]) + be0[...]
        b_ref[...] = _dot(x1, wj[...])

    return pl.pallas_call(
        body,
        grid=(NP // BN,),
        in_specs=[_rows(BN, H)] + _mlp_specs(H, True)
        + [_full((H, H)), _full((H, H)), _full((1, H))],
        out_specs=[_rows(BN, H)] * 3,
        out_shape=[jax.ShapeDtypeStruct((NP, H), jnp.float32)] * 3,
    )(xp, *_mlp_args(ne, True), w0i, w0j, b0e.reshape(1, H))


def _edge_enc_call(eap, ee):
    def body(e_ref, w0, b0, w1, b1, w2, b2, g, bl, out_ref):
        h = _silu(_dot(e_ref[...], w0[...]) + b0[...])
        h = _silu(_dot(h, w1[...]) + b1[...])
        h = _dot(h, w2[...]) + b2[...]
        out_ref[...] = _ln(h, g[...], bl[...])

    return pl.pallas_call(
        body,
        grid=(EPK // BE,),
        in_specs=[_rows(BE, 16)] + _mlp_specs(16, True),
        out_specs=_rows(BE, H),
        out_shape=jax.ShapeDtypeStruct((EPK, H), jnp.float32),
    )(eap, *_mlp_args(ee, True))


def _edge_layer_call(gab, gbb, e, pe):
    def body(ga, gb, e_ref, w0e, w1, b1, w2, b2, g, bl, out_ref):
        ev = e_ref[...]
        h = ga[...] + gb[...] + _dot(ev, w0e[...])
        h = _silu(h)
        h = _silu(_dot(h, w1[...]) + b1[...])
        h = _dot(h, w2[...]) + b2[...]
        out_ref[...] = _ln(h, g[...], bl[...]) + ev

    w0e = pe["l0"]["W"][2 * H:]
    return pl.pallas_call(
        body,
        grid=(EPK // BE,),
        in_specs=[_rows(BE, H), _rows(BE, H), _rows(BE, H), _full((H, H)),
                  _full((H, H)), _full((1, H)), _full((H, H)), _full((1, H)),
                  _full((1, H)), _full((1, H))],
        out_specs=_rows(BE, H),
        out_shape=jax.ShapeDtypeStruct((EPK, H), jnp.float32),
    )(gab, gbb, e, w0e,
      pe["l1"]["W"], pe["l1"]["b"].reshape(1, H),
      pe["l2"]["W"], pe["l2"]["b"].reshape(1, H),
      pe["ln"]["g"].reshape(1, H), pe["ln"]["b"].reshape(1, H))


def _edge_layer0_call(gab, gbb, eap, ee, pe):
    def body(ga, gb, ea_ref, w0, b0, w1, b1, w2, b2, g, bl,
             pw0e, pw1, pb1, pw2, pb2, pg, pbl, out_ref):
        he = _silu(_dot(ea_ref[...], w0[...]) + b0[...])
        he = _silu(_dot(he, w1[...]) + b1[...])
        he = _dot(he, w2[...]) + b2[...]
        ev = _ln(he, g[...], bl[...])
        h = ga[...] + gb[...] + _dot(ev, pw0e[...])
        h = _silu(h)
        h = _silu(_dot(h, pw1[...]) + pb1[...])
        h = _dot(h, pw2[...]) + pb2[...]
        out_ref[...] = _ln(h, pg[...], pbl[...]) + ev

    w0e = pe["l0"]["W"][2 * H:]
    return pl.pallas_call(
        body,
        grid=(EPK // BE,),
        in_specs=[_rows(BE, H), _rows(BE, H), _rows(BE, 16)]
        + _mlp_specs(16, True)
        + [_full((H, H)), _full((H, H)), _full((1, H)), _full((H, H)),
           _full((1, H)), _full((1, H)), _full((1, H))],
        out_specs=_rows(BE, H),
        out_shape=jax.ShapeDtypeStruct((EPK, H), jnp.float32),
    )(gab, gbb, eap, *_mlp_args(ee, True), w0e,
      pe["l1"]["W"], pe["l1"]["b"].reshape(1, H),
      pe["l2"]["W"], pe["l2"]["b"].reshape(1, H),
      pe["ln"]["g"].reshape(1, H), pe["ln"]["b"].reshape(1, H))


def _node_layer_call(x1, s2, pn, w0i, w0j, b0e):
    def body(x_ref, s0, s1, s2, s3, w0x, w0a, b0, w1, b1, w2, b2, g, bl,
             wi, wj, be0, x_out, a_out, b_out):
        x = x_ref[...]
        agg = (s0[...] + s1[...]) + (s2[...] + s3[...])
        h = _silu(_dot(x, w0x[...]) + _dot(agg, w0a[...]) + b0[...])
        h = _silu(_dot(h, w1[...]) + b1[...])
        h = _dot(h, w2[...]) + b2[...]
        xn = x + _ln(h, g[...], bl[...])
        x_out[...] = xn
        a_out[...] = _dot(xn, wi[...]) + be0[...]
        b_out[...] = _dot(xn, wj[...])

    w0 = pn["l0"]["W"]
    nb = NP // BN
    return pl.pallas_call(
        body,
        grid=(nb,),
        in_specs=[_rows(BN, H),
                  pl.BlockSpec((BN, H), lambda i: (i, 0)),
                  pl.BlockSpec((BN, H), lambda i: (i + nb, 0)),
                  pl.BlockSpec((BN, H), lambda i: (i, 0)),
                  pl.BlockSpec((BN, H), lambda i: (i + nb, 0)),
                  _full((H, H)), _full((H, H)), _full((1, H)),
                  _full((H, H)), _full((1, H)), _full((H, H)), _full((1, H)),
                  _full((1, H)), _full((1, H)),
                  _full((H, H)), _full((H, H)), _full((1, H))],
        out_specs=[_rows(BN, H)] * 3,
        out_shape=[jax.ShapeDtypeStruct((NP, H), jnp.float32)] * 3,
    )(x1, s2[0], s2[0], s2[1], s2[1], w0[:H], w0[H:],
      pn["l0"]["b"].reshape(1, H),
      pn["l1"]["W"], pn["l1"]["b"].reshape(1, H),
      pn["l2"]["W"], pn["l2"]["b"].reshape(1, H),
      pn["ln"]["g"].reshape(1, H), pn["ln"]["b"].reshape(1, H),
      w0i, w0j, b0e.reshape(1, H))


def _final_call(x1, s2, pn, dec):
    def body(x_ref, s0, s1, s2, s3, w0x, w0a, b0, w1, b1, w2, b2, g, bl,
             d0, db0, d1, db1, d2, db2, out_ref):
        x = x_ref[...]
        agg = (s0[...] + s1[...]) + (s2[...] + s3[...])
        h = _silu(_dot(x, w0x[...]) + _dot(agg, w0a[...]) + b0[...])
        h = _silu(_dot(h, w1[...]) + b1[...])
        h = _dot(h, w2[...]) + b2[...]
        xn = x + _ln(h, g[...], bl[...])
        h = _silu(_dot(xn, d0[...]) + db0[...])
        h = _silu(_dot(h, d1[...]) + db1[...])
        out_ref[...] = _dot(h, d2[...]) + db2[...]

    w0 = pn["l0"]["W"]
    nb = NP // BN
    return pl.pallas_call(
        body,
        grid=(nb,),
        in_specs=[_rows(BN, H),
                  pl.BlockSpec((BN, H), lambda i: (i, 0)),
                  pl.BlockSpec((BN, H), lambda i: (i + nb, 0)),
                  pl.BlockSpec((BN, H), lambda i: (i, 0)),
                  pl.BlockSpec((BN, H), lambda i: (i + nb, 0)),
                  _full((H, H)), _full((H, H)), _full((1, H)),
                  _full((H, H)), _full((1, H)), _full((H, H)), _full((1, H)),
                  _full((1, H)), _full((1, H))] + _mlp_specs(H, False),
        out_specs=_rows(BN, H),
        out_shape=jax.ShapeDtypeStruct((NP, H), jnp.float32),
    )(x1, s2[0], s2[0], s2[1], s2[1], w0[:H], w0[H:],
      pn["l0"]["b"].reshape(1, H),
      pn["l1"]["W"], pn["l1"]["b"].reshape(1, H),
      pn["l2"]["W"], pn["l2"]["b"].reshape(1, H),
      pn["ln"]["g"].reshape(1, H), pn["ln"]["b"].reshape(1, H),
      *_mlp_args(dec, False))


def _sc_gather(ai, bi, dstc, srcc):
    mesh = plsc.VectorSubcoreMesh(core_axis_name="c", subcore_axis_name="s")
    pw = EPK // NS
    nch = pw // CHUNK

    @functools.partial(
        pl.kernel,
        out_type=(jax.ShapeDtypeStruct((EPK, H), jnp.float32),) * 2,
        mesh=mesh,
        scratch_types=[
            pltpu.VMEM_SHARED((NP, H), jnp.float32),
            pltpu.VMEM((nch, CHUNK), jnp.int32),
            pltpu.VMEM((2, CHUNK, H), jnp.float32),
            pltpu.SemaphoreType.DMA,
            pltpu.SemaphoreType.DMA,
        ],
    )
    def k(a_hbm, b_hbm, d_hbm, s_hbm, ga_hbm, gb_hbm,
          table, i_v, buf, semg, semw):
        c = lax.axis_index("c")
        s = lax.axis_index("s")

        def side(t_hbm, i_hbm, o_hbm):
            pltpu.sync_copy(t_hbm.at[pl.ds(s * RSUB, RSUB)],
                            table.at[pl.ds(s * RSUB, RSUB)])
            pltpu.sync_copy(i_hbm.at[s], i_v)
            plsc.subcore_barrier()
            base = s * pw

            def row(j):
                return o_hbm.at[pl.ds(base + j * CHUNK, CHUNK)]

            @pl.loop(0, nch, step=2)
            def _(j0):
                for t in range(2):
                    j = j0 + t

                    @pl.when(j0 >= 2)
                    def _():
                        pltpu.make_async_copy(buf.at[t], row(j - 2),
                                              semw).wait()

                    cg = pltpu.async_copy(table.at[i_v.at[j]], buf.at[t],
                                          semg)
                    cg.wait()
                    pltpu.async_copy(buf.at[t], row(j), semw)

            for t in range(2):
                j = nch - 2 + t
                pltpu.make_async_copy(buf.at[t], row(j), semw).wait()

        @pl.when(c == 0)
        def _():
            side(a_hbm, d_hbm, ga_hbm)

        @pl.when(c == 1)
        def _():
            side(b_hbm, s_hbm, gb_hbm)

    return k(ai, bi, dstc, srcc)


def _sc_scatter(e_new, srcc, zsub):
    mesh = plsc.VectorSubcoreMesh(core_axis_name="c", subcore_axis_name="s")

    @functools.partial(
        pl.kernel,
        out_type=jax.ShapeDtypeStruct((2 * NP, H), jnp.float32),
        mesh=mesh,
        scratch_types=[
            pltpu.VMEM_SHARED((NP, H), jnp.float32),
            pltpu.VMEM((NCHK, CHUNK), jnp.int32),
            pltpu.VMEM((2, CHUNK, H), jnp.float32),
            pltpu.SemaphoreType.DMA,
            pltpu.SemaphoreType.DMA,
        ],
    )
    def k(e_hbm, s_hbm, z_hbm, out_hbm, shared, s_v, buf, semr, sema):
        c = lax.axis_index("c")
        s = lax.axis_index("s")
        wid = s * NC + c
        pltpu.sync_copy(z_hbm, shared.at[pl.ds(s * RSUB, RSUB)])
        pltpu.sync_copy(s_hbm.at[wid], s_v)
        plsc.subcore_barrier()
        base = wid * PER_WK

        @pl.loop(0, NCHK, step=2)
        def _(j0):
            for t in range(2):
                j = j0 + t

                @pl.when(j0 >= 2)
                def _():
                    pltpu.make_async_copy(buf.at[t], shared.at[s_v.at[j - 2]],
                                          sema).wait()

                cr = pltpu.async_copy(
                    e_hbm.at[pl.ds(base + j * CHUNK, CHUNK)], buf.at[t], semr)
                cr.wait()
                pltpu.async_copy(buf.at[t], shared.at[s_v.at[j]], sema,
                                 add=True)

        for t in range(2):
            j = NCHK - 2 + t
            pltpu.make_async_copy(buf.at[t], shared.at[s_v.at[j]], sema).wait()

        plsc.subcore_barrier()
        pltpu.sync_copy(shared.at[pl.ds(s * RSUB, RSUB)],
                        out_hbm.at[pl.ds(c * NP + s * RSUB, RSUB)])

    return k(e_new, srcc, zsub)


def kernel(x, edge_index, edge_attr, params):
    src = edge_index[0].astype(jnp.int32)
    dst = edge_index[1].astype(jnp.int32)
    pad_e = EP - E
    pad_idx = N + jnp.arange(pad_e, dtype=jnp.int32) % (NP - N)
    srcp = jnp.concatenate([src, pad_idx]).reshape(K, NW, NCHK, CHUNK)
    dstp = jnp.concatenate([dst, pad_idx]).reshape(K, NW, NCHK, CHUNK)
    srcc = [srcp[k] for k in range(K)]
    dstc = [dstp[k] for k in range(K)]
    srcg = [srcp[k].reshape(NS, EPK // NS // CHUNK, CHUNK) for k in range(K)]
    dstg = [dstp[k].reshape(NS, EPK // NS // CHUNK, CHUNK) for k in range(K)]
    xp = jnp.pad(x, ((0, NP - N), (0, 0)))
    eap = jnp.pad(edge_attr, ((0, pad_e), (0, 0))).reshape(K, EPK, 16)
    zsub = jnp.zeros((RSUB, H), jnp.float32)

    pr = params["proc"]
    w00 = pr[0]["edge_mlp"]["l0"]["W"]
    x1, at, bt = _node_enc_call(xp, params["node_enc"], w00[:H], w00[H:2 * H],
                                pr[0]["edge_mlp"]["l0"]["b"])
    e = [None] * K

    out = None
    for i in range(3):
        pe = pr[i]["edge_mlp"]
        s2 = []
        for k in range(K):
            ga, gb = _sc_gather(at, bt, dstg[k], srcg[k])
            if i == 0:
                e[k] = _edge_layer0_call(ga, gb, eap[k], params["edge_enc"],
                                         pe)
            else:
                e[k] = _edge_layer_call(ga, gb, e[k], pe)
            s2.append(_sc_scatter(e[k], srcc[k], zsub))
        if i < 2:
            w0n = pr[i + 1]["edge_mlp"]["l0"]["W"]
            x1, at, bt = _node_layer_call(
                x1, s2, pr[i]["node_mlp"], w0n[:H], w0n[H:2 * H],
                pr[i + 1]["edge_mlp"]["l0"]["b"])
        else:
            out = _final_call(x1, s2, pr[i]["node_mlp"], params["decoder"])
    return out[:N]
